# baseline (speedup 1.0000x reference)
import jax
import jax.numpy as jnp
from jax import lax
from jax.experimental import pallas as pl
from jax.experimental.pallas import tpu as pltpu
from jax.experimental.pallas import tpu_sc as plsc

VOCAB = 1000000
N_EMBD = 64
SEQ = 200
BATCH = 4096

NC, NS = 2, 16
NW = NC * NS
NCOLS = (VOCAB + 127) // 128
K1_ITERS = (NCOLS + NW - 1) // NW
NT8 = SEQ // 8
NBC = BATCH // 128


def _iota16():
    return lax.iota(jnp.int32, 16)


def _splat16(v):
    return jnp.zeros((16,), jnp.int32) + v


def _k1_body(ttok, ttail, out, in_v, out_v):
    wid = lax.axis_index("s") * NC + lax.axis_index("c")

    def col_body(i, carry):
        col = wid + i * NW
        last = NCOLS - 1

        @pl.when(col < last)
        def _full():
            pltpu.sync_copy(ttok.at[:, pl.ds(col * 128, 128)], in_v)
            for p in range(64):
                for k in range(8):
                    l = 2 * p + (1 if k >= 4 else 0)
                    e0 = 16 * (k % 4)
                    vec = plsc.load_gather(
                        in_v, [_iota16() + e0, _splat16(l)])
                    out_v[p, pl.ds(e0 + (64 if k >= 4 else 0), 16)] = vec
            pltpu.sync_copy(out_v, out.at[pl.ds(col * 64, 64)])

        @pl.when(col == last)
        def _tail():
            pltpu.sync_copy(ttail, in_v)
            for p in range(32):
                for k in range(8):
                    l = 2 * p + (1 if k >= 4 else 0)
                    e0 = 16 * (k % 4)
                    vec = plsc.load_gather(
                        in_v, [_iota16() + e0, _splat16(l)])
                    out_v[p, pl.ds(e0 + (64 if k >= 4 else 0), 16)] = vec
            pltpu.sync_copy(out_v.at[pl.ds(0, 32)],
                            out.at[pl.ds(col * 64, 32)])

        return carry

    lax.fori_loop(0, K1_ITERS, col_body, 0)


def _k2_body(xt, posp, table, out,
             idx_v, idx2_v, par_v, pos_v, rows_v, out_v, sem):
    wid = lax.axis_index("s") * NC + lax.axis_index("c")
    bc = wid

    def block_body(t8, carry):
        pltpu.sync_copy(xt.at[pl.ds(t8 * 8, 8), pl.ds(bc * 128, 128)], idx_v)
        pltpu.sync_copy(posp.at[pl.ds(t8 * 8, 8)], pos_v)
        for r in range(8):
            for m in range(8):
                v = idx_v[r, pl.ds(16 * m, 16)]
                idx2_v[r, pl.ds(16 * m, 16)] = lax.shift_right_logical(v, 1)
                par_v[r, pl.ds(16 * m, 16)] = lax.shift_left(
                    lax.bitwise_and(v, 1), 6)

        def t_body(tl, carry2):
            pltpu.async_copy(table.at[idx2_v.at[tl]], rows_v, sem).wait()
            par = [par_v[tl, pl.ds(16 * m, 16)] for m in range(8)]
            for er in range(8):
                for es in range(8):
                    e = 8 * er + es
                    pspl = plsc.load_gather(
                        pos_v, [_splat16(tl), _splat16(e)])
                    for m in range(8):
                        rvec = plsc.load_gather(
                            rows_v, [_iota16() + 16 * m, par[m] + e])
                        out_v[er, es, pl.ds(16 * m, 16)] = rvec + pspl
            for er in range(8):
                pltpu.sync_copy(out_v.at[er], out.at[t8 * 8 + tl, er, bc])
            return carry2

        lax.fori_loop(0, 8, t_body, 0)
        return carry

    lax.fori_loop(0, NT8, block_body, 0)


def kernel(x, tok_emb, pos_emb):
    mesh = plsc.VectorSubcoreMesh(core_axis_name="c", subcore_axis_name="s")
    params = pltpu.CompilerParams(use_tc_tiling_on_sc=True,
                                  needs_layout_passes=False)

    k1 = pl.kernel(
        _k1_body,
        out_type=jax.ShapeDtypeStruct((VOCAB // 2, 128), jnp.float32),
        mesh=mesh,
        compiler_params=params,
        scratch_types=[
            pltpu.VMEM((64, 128), jnp.float32),
            pltpu.VMEM((64, 128), jnp.float32),
        ],
    )
    k2 = pl.kernel(
        _k2_body,
        out_type=jax.ShapeDtypeStruct((SEQ, 8, NBC, 8, 128), jnp.float32),
        mesh=mesh,
        compiler_params=params,
        scratch_types=[
            pltpu.VMEM((8, 128), jnp.int32),
            pltpu.VMEM((8, 128), jnp.int32),
            pltpu.VMEM((8, 128), jnp.int32),
            pltpu.VMEM((8, 128), jnp.float32),
            pltpu.VMEM((128, 128), jnp.float32),
            pltpu.VMEM((8, 8, 128), jnp.float32),
            pltpu.SemaphoreType.DMA,
        ],
    )

    ttok = tok_emb.T
    ttail = jnp.pad(tok_emb[VOCAB - 64:].T, ((0, 0), (0, 64)))
    table = k1(ttok, ttail)
    xt = x.astype(jnp.int32).T
    posp = jnp.pad(pos_emb[:SEQ], ((0, 0), (0, 64)))
    o5 = k2(xt, posp, table)
    return o5.transpose(2, 4, 0, 1, 3).reshape(BATCH, SEQ, N_EMBD)


# speedup vs baseline: 1.5219x; 1.5219x over previous
import jax
import jax.numpy as jnp
from jax import lax
from jax.experimental import pallas as pl
from jax.experimental.pallas import tpu as pltpu
from jax.experimental.pallas import tpu_sc as plsc

VOCAB = 1000000
N_EMBD = 64
SEQ = 200
BATCH = 4096

NC, NS = 2, 16
NW = NC * NS
NCOLS = (VOCAB + 127) // 128
K1_PER_W = 245
K1_PAIRS = 123
NPAIR = VOCAB // 2 + 32
NT8 = SEQ // 8
NBC = BATCH // 128


def _iota16():
    return lax.iota(jnp.int32, 16)


def _splat16(v):
    return jnp.zeros((16,), jnp.int32) + v


def _k1_body(ttok, ttail, out,
             in_a, in_b, out_a, out_b, isem_a, isem_b, wsem_a, wsem_b):
    wid = lax.axis_index("s") * NC + lax.axis_index("c")
    base = wid * K1_PER_W
    last = NCOLS - 1

    def start_in(col, buf, sem):
        @pl.when(col < last)
        def _full():
            pltpu.async_copy(ttok.at[:, pl.ds(col * 128, 128)], buf, sem)

        @pl.when(col == last)
        def _tail():
            pltpu.async_copy(ttail, buf, sem)

    def wait_in(buf, sem):
        pltpu.make_async_copy(ttok.at[:, pl.ds(0, 128)], buf, sem).wait()

    def transpose(in_v, out_v):
        def c_body(c, carry):
            lvec = _iota16() + 16 * c
            pvec = lax.shift_right_logical(lvec, 1)
            jbase = lax.shift_left(lax.bitwise_and(lvec, 1), 6)
            for e in range(64):
                x = in_v[e, pl.ds(c * 16, 16)]
                plsc.store_scatter(out_v, [pvec, jbase + e], x)
            return carry

        lax.fori_loop(0, 8, c_body, 0)

    def write_out(col, out_v, wsem):
        pltpu.async_copy(out_v.at[:, pl.ds(0, 128)],
                         out.at[pl.ds(col * 64, 64)], wsem)

    def drain_out(out_v, wsem):
        pltpu.make_async_copy(out_v.at[:, pl.ds(0, 128)],
                              out.at[pl.ds(0, 64)], wsem).wait()

    def ce(i):
        return jnp.minimum(base + i, last)

    start_in(ce(0), in_a, isem_a)

    def pair_body(i, carry):
        ca = ce(2 * i)
        cb = ce(2 * i + 1)
        cn = ce(2 * i + 2)
        wait_in(in_a, isem_a)
        start_in(cb, in_b, isem_b)

        @pl.when(i > 0)
        def _da():
            drain_out(out_a, wsem_a)

        transpose(in_a, out_a)
        write_out(ca, out_a, wsem_a)
        wait_in(in_b, isem_b)

        @pl.when(i < K1_PAIRS - 1)
        def _na():
            start_in(cn, in_a, isem_a)

        @pl.when(i > 0)
        def _db():
            drain_out(out_b, wsem_b)

        transpose(in_b, out_b)
        write_out(cb, out_b, wsem_b)
        return carry

    lax.fori_loop(0, K1_PAIRS, pair_body, 0)
    drain_out(out_a, wsem_a)
    drain_out(out_b, wsem_b)


def _k2_body(xt, posp, table, out,
             idx_v, idx2_v, pos_v, rows_a, rows_b, out_a, out_b,
             gsem_a, gsem_b, wsem_a, wsem_b):
    wid = lax.axis_index("s") * NC + lax.axis_index("c")
    bc = wid

    def start_g(tl, rows, sem):
        pltpu.async_copy(table.at[idx2_v.at[tl]], rows, sem)

    def wait_g(rows, sem):
        pltpu.make_async_copy(table.at[idx2_v.at[0]], rows, sem).wait()

    def proc(tl, rows_v, out_v):
        pks = [pos_v[tl, pl.ds(16 * k, 16)] for k in range(4)]
        evecs = [_iota16() + 16 * k for k in range(4)]
        def m_body(m, carry):
            vm = idx_v[tl, pl.ds(m * 16, 16)]
            b0 = m * 16
            for q in range(16):
                off = lax.shift_left(lax.bitwise_and(vm[q], 1), 6)
                bvec = _splat16(b0 + q)
                for k in range(4):
                    x = rows_v[b0 + q, pl.ds(off + 16 * k, 16)]
                    plsc.store_scatter(out_v, [evecs[k], bvec], x + pks[k])
            return carry

        lax.fori_loop(0, 8, m_body, 0)

    def write_o(t_abs, out_v, wsem):
        for er in range(8):
            pltpu.async_copy(out_v.at[pl.ds(er * 8, 8), pl.ds(0, 128)],
                             out.at[t_abs, er, bc], wsem)

    def drain_w(out_v, wsem):
        for er in range(8):
            pltpu.make_async_copy(out_v.at[pl.ds(0, 8), pl.ds(0, 128)],
                                  out.at[0, 0, bc], wsem).wait()

    def block(t8, carry):
        pltpu.sync_copy(xt.at[pl.ds(t8 * 8, 8), pl.ds(bc * 128, 128)], idx_v)
        pltpu.sync_copy(posp.at[pl.ds(t8 * 8, 8)], pos_v)
        for r in range(8):
            for m in range(8):
                idx2_v[r, pl.ds(16 * m, 16)] = lax.shift_right_logical(
                    idx_v[r, pl.ds(16 * m, 16)], 1)
        start_g(0, rows_a, gsem_a)

        def tp_body(tp, c2):
            ta = 2 * tp
            tb = 2 * tp + 1
            not_first = jnp.logical_or(t8 > 0, tp > 0)
            wait_g(rows_a, gsem_a)
            start_g(tb, rows_b, gsem_b)

            @pl.when(not_first)
            def _da():
                drain_w(out_a, wsem_a)

            proc(ta, rows_a, out_a)
            write_o(t8 * 8 + ta, out_a, wsem_a)
            wait_g(rows_b, gsem_b)

            @pl.when(tp < 3)
            def _ng():
                start_g(ta + 2, rows_a, gsem_a)

            @pl.when(not_first)
            def _db():
                drain_w(out_b, wsem_b)

            proc(tb, rows_b, out_b)
            write_o(t8 * 8 + tb, out_b, wsem_b)
            return c2

        lax.fori_loop(0, 4, tp_body, 0)
        return carry

    lax.fori_loop(0, NT8, block, 0)
    drain_w(out_a, wsem_a)
    drain_w(out_b, wsem_b)


def kernel(x, tok_emb, pos_emb):
    mesh = plsc.VectorSubcoreMesh(core_axis_name="c", subcore_axis_name="s")
    params = pltpu.CompilerParams(use_tc_tiling_on_sc=True,
                                  needs_layout_passes=False)

    k1 = pl.kernel(
        _k1_body,
        out_type=jax.ShapeDtypeStruct((NPAIR, 128), jnp.float32),
        mesh=mesh,
        compiler_params=params,
        scratch_types=[
            pltpu.VMEM((64, 128), jnp.float32),
            pltpu.VMEM((64, 128), jnp.float32),
            pltpu.VMEM((64, 129), jnp.float32),
            pltpu.VMEM((64, 129), jnp.float32),
            pltpu.SemaphoreType.DMA,
            pltpu.SemaphoreType.DMA,
            pltpu.SemaphoreType.DMA,
            pltpu.SemaphoreType.DMA,
        ],
    )
    k2 = pl.kernel(
        _k2_body,
        out_type=jax.ShapeDtypeStruct((SEQ, 8, NBC, 8, 128), jnp.float32),
        mesh=mesh,
        compiler_params=params,
        scratch_types=[
            pltpu.VMEM((8, 128), jnp.int32),
            pltpu.VMEM((8, 128), jnp.int32),
            pltpu.VMEM((8, 128), jnp.float32),
            pltpu.VMEM((128, 128), jnp.float32),
            pltpu.VMEM((128, 128), jnp.float32),
            pltpu.VMEM((64, 129), jnp.float32),
            pltpu.VMEM((64, 129), jnp.float32),
            pltpu.SemaphoreType.DMA,
            pltpu.SemaphoreType.DMA,
            pltpu.SemaphoreType.DMA,
            pltpu.SemaphoreType.DMA,
        ],
    )

    ttok = tok_emb.T
    ttail = jnp.pad(tok_emb[VOCAB - 64:].T, ((0, 0), (0, 64)))
    table = k1(ttok, ttail)
    xt = x.astype(jnp.int32).T
    posp = jnp.pad(pos_emb[:SEQ], ((0, 0), (0, 64)))
    o5 = k2(xt, posp, table)
    return o5.transpose(2, 4, 0, 1, 3).reshape(BATCH, SEQ, N_EMBD)


# speedup vs baseline: 1.9641x; 1.2906x over previous
import jax
import jax.numpy as jnp
from jax import lax
from jax.experimental import pallas as pl
from jax.experimental.pallas import tpu as pltpu
from jax.experimental.pallas import tpu_sc as plsc

VOCAB = 1000000
N_EMBD = 64
SEQ = 200
BATCH = 4096

NC, NS = 2, 16
NW = NC * NS
NCOLS = (VOCAB + 127) // 128
K1_PER_W = 245
K1_PAIRS = 123
NPAIR = VOCAB // 2 + 32
NT8 = SEQ // 8
NBC = BATCH // 128


def _iota16():
    return lax.iota(jnp.int32, 16)


def _splat16(v):
    return jnp.zeros((16,), jnp.int32) + v


def _k1_body(ttok, ttail, out,
             in_a, in_b, out_a, out_b, isem_a, isem_b, wsem_a, wsem_b):
    wid = lax.axis_index("s") * NC + lax.axis_index("c")
    base = wid * K1_PER_W
    last = NCOLS - 1

    def start_in(col, buf, sem):
        @pl.when(col < last)
        def _full():
            pltpu.async_copy(ttok.at[:, pl.ds(col * 128, 128)], buf, sem)

        @pl.when(col == last)
        def _tail():
            pltpu.async_copy(ttail, buf, sem)

    def wait_in(buf, sem):
        pltpu.make_async_copy(ttok.at[:, pl.ds(0, 128)], buf, sem).wait()

    def transpose(in_v, out_v):
        @plsc.parallel_loop(0, 8)
        def c_body(c):
            lvec = _iota16() + 16 * c
            pvec = lax.shift_right_logical(lvec, 1)
            jbase = lax.shift_left(lax.bitwise_and(lvec, 1), 6)
            for e in range(64):
                x = in_v[e, pl.ds(c * 16, 16)]
                plsc.store_scatter(out_v, [pvec, jbase + e], x)

    def write_out(col, out_v, wsem):
        pltpu.async_copy(out_v.at[:, pl.ds(0, 128)],
                         out.at[pl.ds(col * 64, 64)], wsem)

    def drain_out(out_v, wsem):
        pltpu.make_async_copy(out_v.at[:, pl.ds(0, 128)],
                              out.at[pl.ds(0, 64)], wsem).wait()

    def ce(i):
        return jnp.minimum(base + i, last)

    start_in(ce(0), in_a, isem_a)

    def pair_body(i, carry):
        ca = ce(2 * i)
        cb = ce(2 * i + 1)
        cn = ce(2 * i + 2)
        wait_in(in_a, isem_a)
        start_in(cb, in_b, isem_b)

        @pl.when(i > 0)
        def _da():
            drain_out(out_a, wsem_a)

        transpose(in_a, out_a)
        write_out(ca, out_a, wsem_a)
        wait_in(in_b, isem_b)

        @pl.when(i < K1_PAIRS - 1)
        def _na():
            start_in(cn, in_a, isem_a)

        @pl.when(i > 0)
        def _db():
            drain_out(out_b, wsem_b)

        transpose(in_b, out_b)
        write_out(cb, out_b, wsem_b)
        return carry

    lax.fori_loop(0, K1_PAIRS, pair_body, 0)
    drain_out(out_a, wsem_a)
    drain_out(out_b, wsem_b)


def _k2_body(xt, posp, table, out,
             idx_v, idx2_v, pos_v, rows_a, rows_b, out_a, out_b,
             gsem_a, gsem_b, wsem_a, wsem_b):
    wid = lax.axis_index("s") * NC + lax.axis_index("c")
    bc = wid

    def start_g(tl, rows, sem):
        pltpu.async_copy(table.at[idx2_v.at[tl]], rows, sem)

    def wait_g(rows, sem):
        pltpu.make_async_copy(table.at[idx2_v.at[0]], rows, sem).wait()

    def proc(tl, rows_v, out_v):
        pks = [pos_v[tl, pl.ds(16 * k, 16)] for k in range(4)]
        evecs = [_iota16() + 16 * k for k in range(4)]
        @plsc.parallel_loop(0, 8)
        def m_body(m):
            vm = idx_v[tl, pl.ds(m * 16, 16)]
            b0 = m * 16
            for q in range(16):
                off = lax.shift_left(lax.bitwise_and(vm[q], 1), 6)
                bvec = _splat16(b0 + q)
                for k in range(4):
                    x = rows_v[b0 + q, pl.ds(off + 16 * k, 16)]
                    plsc.store_scatter(out_v, [evecs[k], bvec], x + pks[k])

    def write_o(t_abs, out_v, wsem):
        for er in range(8):
            pltpu.async_copy(out_v.at[pl.ds(er * 8, 8), pl.ds(0, 128)],
                             out.at[t_abs, er, bc], wsem)

    def drain_w(out_v, wsem):
        for er in range(8):
            pltpu.make_async_copy(out_v.at[pl.ds(0, 8), pl.ds(0, 128)],
                                  out.at[0, 0, bc], wsem).wait()

    def block(t8, carry):
        pltpu.sync_copy(xt.at[pl.ds(t8 * 8, 8), pl.ds(bc * 128, 128)], idx_v)
        pltpu.sync_copy(posp.at[pl.ds(t8 * 8, 8)], pos_v)
        for r in range(8):
            for m in range(8):
                idx2_v[r, pl.ds(16 * m, 16)] = lax.shift_right_logical(
                    idx_v[r, pl.ds(16 * m, 16)], 1)
        start_g(0, rows_a, gsem_a)

        def tp_body(tp, c2):
            ta = 2 * tp
            tb = 2 * tp + 1
            not_first = jnp.logical_or(t8 > 0, tp > 0)
            wait_g(rows_a, gsem_a)
            start_g(tb, rows_b, gsem_b)

            @pl.when(not_first)
            def _da():
                drain_w(out_a, wsem_a)

            proc(ta, rows_a, out_a)
            write_o(t8 * 8 + ta, out_a, wsem_a)
            wait_g(rows_b, gsem_b)

            @pl.when(tp < 3)
            def _ng():
                start_g(ta + 2, rows_a, gsem_a)

            @pl.when(not_first)
            def _db():
                drain_w(out_b, wsem_b)

            proc(tb, rows_b, out_b)
            write_o(t8 * 8 + tb, out_b, wsem_b)
            return c2

        lax.fori_loop(0, 4, tp_body, 0)
        return carry

    lax.fori_loop(0, NT8, block, 0)
    drain_w(out_a, wsem_a)
    drain_w(out_b, wsem_b)


def kernel(x, tok_emb, pos_emb):
    mesh = plsc.VectorSubcoreMesh(core_axis_name="c", subcore_axis_name="s")
    params = pltpu.CompilerParams(use_tc_tiling_on_sc=True,
                                  needs_layout_passes=False)

    k1 = pl.kernel(
        _k1_body,
        out_type=jax.ShapeDtypeStruct((NPAIR, 128), jnp.float32),
        mesh=mesh,
        compiler_params=params,
        scratch_types=[
            pltpu.VMEM((64, 128), jnp.float32),
            pltpu.VMEM((64, 128), jnp.float32),
            pltpu.VMEM((64, 129), jnp.float32),
            pltpu.VMEM((64, 129), jnp.float32),
            pltpu.SemaphoreType.DMA,
            pltpu.SemaphoreType.DMA,
            pltpu.SemaphoreType.DMA,
            pltpu.SemaphoreType.DMA,
        ],
    )
    k2 = pl.kernel(
        _k2_body,
        out_type=jax.ShapeDtypeStruct((SEQ, 8, NBC, 8, 128), jnp.float32),
        mesh=mesh,
        compiler_params=params,
        scratch_types=[
            pltpu.VMEM((8, 128), jnp.int32),
            pltpu.VMEM((8, 128), jnp.int32),
            pltpu.VMEM((8, 128), jnp.float32),
            pltpu.VMEM((128, 128), jnp.float32),
            pltpu.VMEM((128, 128), jnp.float32),
            pltpu.VMEM((64, 129), jnp.float32),
            pltpu.VMEM((64, 129), jnp.float32),
            pltpu.SemaphoreType.DMA,
            pltpu.SemaphoreType.DMA,
            pltpu.SemaphoreType.DMA,
            pltpu.SemaphoreType.DMA,
        ],
    )

    ttok = tok_emb.T
    ttail = jnp.pad(tok_emb[VOCAB - 64:].T, ((0, 0), (0, 64)))
    table = k1(ttok, ttail)
    xt = x.astype(jnp.int32).T
    posp = jnp.pad(pos_emb[:SEQ], ((0, 0), (0, 64)))
    o5 = k2(xt, posp, table)
    return o5.transpose(2, 4, 0, 1, 3).reshape(BATCH, SEQ, N_EMBD)
